# baseline (speedup 1.0000x reference)
import functools

import jax
import jax.numpy as jnp
from jax import lax
from jax.experimental import pallas as pl
from jax.experimental.pallas import tpu as pltpu

B, C, N, FIN = 4, 5, 512, 128
H, D = 4, 64
HD = H * D
OUT = 16
NEG = -1e9
_PREC = lax.Precision.HIGHEST


def _gat_core(h, adjm, vc, vr, W, Al, ArT, brow, Wp1, bp1, Wp2):
    feat = jnp.dot(h, W, precision=_PREC)
    el = jnp.dot(feat, Al, precision=_PREC)
    ert = lax.dot_general(ArT, feat, (((1,), (1,)), ((), ())),
                          precision=_PREC)
    cond = jnp.logical_and(adjm != 0.0,
                           jnp.logical_and(vc > 0.0, vr > 0.0))
    outs = []
    for hh in range(H):
        s = el[:, hh:hh + 1] + ert[hh:hh + 1, :]
        s = jnp.where(s >= 0.0, s, 0.2 * s)
        s = jnp.where(cond, s, NEG)
        m = jnp.max(s, axis=0, keepdims=True)
        p = jnp.exp(s - m)
        denom = jnp.sum(p, axis=0, keepdims=True)
        p = p / denom
        outs.append(lax.dot_general(p, feat[:, hh * D:(hh + 1) * D],
                                    (((0,), (0,)), ((), ())),
                                    precision=_PREC))
    rst = jnp.concatenate(outs, axis=1) + brow
    z = jnp.where(rst > 0.0, rst, jnp.exp(rst) - 1.0)
    p1 = jnp.tanh(jnp.dot(z, Wp1, precision=_PREC) + bp1)
    pr = jnp.dot(p1, Wp2, precision=_PREC)
    s_partial = jnp.sum(pr * vc)
    return z, s_partial


def _k1_body(h_ref, adj_ref, vc_ref, vr_ref, w_ref, al_ref, art_ref, b_ref,
             wp1_ref, bp1_ref, wp2_ref, z_ref, s_ref):
    z, s = _gat_core(h_ref[0], adj_ref[0, 0], vc_ref[0], vr_ref[0],
                     w_ref[0], al_ref[0], art_ref[0], b_ref[0],
                     wp1_ref[...], bp1_ref[...], wp2_ref[...])
    z_ref[0, 0] = z
    s_ref[0, 0, 0, :] = jnp.full((128,), s, dtype=jnp.float32)


def _k2_body(z0_ref, beta_ref, adj_ref, vc_ref, vr_ref, w_ref, al_ref,
             art_ref, b_ref, wp1_ref, bp1_ref, wp2_ref, z_ref, s_ref, h_scr):
    @pl.when(pl.program_id(1) == 0)
    def _combine():
        acc = beta_ref[0] * z0_ref[0, 0]
        for cc in range(1, C):
            acc = acc + beta_ref[cc] * z0_ref[0, cc]
        h_scr[...] = acc

    z, s = _gat_core(h_scr[...], adj_ref[0, 0], vc_ref[0], vr_ref[0],
                     w_ref[0], al_ref[0], art_ref[0], b_ref[0],
                     wp1_ref[...], bp1_ref[...], wp2_ref[...])
    z_ref[0, 0] = z
    s_ref[0, 0, 0, :] = jnp.full((128,), s, dtype=jnp.float32)


def _k3_body(z1_ref, beta_ref, wpred_ref, bpred_ref, vc_ref, out_ref):
    acc = beta_ref[0] * z1_ref[0, 0]
    for cc in range(1, C):
        acc = acc + beta_ref[cc] * z1_ref[0, cc]
    res = jnp.dot(acc, wpred_ref[...], precision=_PREC) + bpred_ref[...]
    out_ref[0] = res * vc_ref[0]


def _expand_attn(a):
    eye = jnp.eye(H, dtype=a.dtype)
    m = a[:, :, :, None] * eye[None, :, None, :]
    return m.reshape(C, HD, H)


def kernel(x, adj, node_nums, W0, al0, ar0, b0, Wp1_0, bp1_0, Wp2_0,
           W1, al1, ar1, b1, Wp1_1, bp1_1, Wp2_1, Wpred, bpred):
    f32 = jnp.float32
    h0 = x[:, 0]
    ar = jnp.arange(N, dtype=jnp.int32)
    valid = (ar[None, :] < node_nums[:, None]).astype(f32)
    vc = valid[:, :, None]
    vr = valid[:, None, :]
    cnt = jnp.maximum(jnp.sum(valid), 1.0)

    al0m, ar0t = _expand_attn(al0), _expand_attn(ar0).transpose(0, 2, 1)
    al1m, ar1t = _expand_attn(al1), _expand_attn(ar1).transpose(0, 2, 1)
    b0r, b1r = b0.reshape(C, 1, HD), b1.reshape(C, 1, HD)
    bp1_0r, bp1_1r = bp1_0.reshape(1, 128), bp1_1.reshape(1, 128)
    bpredr = bpred.reshape(1, OUT)

    spec_adj = pl.BlockSpec((1, 1, N, N), lambda b, c: (b, c, 0, 0))
    spec_vc = pl.BlockSpec((1, N, 1), lambda b, c: (b, 0, 0))
    spec_vr = pl.BlockSpec((1, 1, N), lambda b, c: (b, 0, 0))
    spec_al = pl.BlockSpec((1, HD, H), lambda b, c: (c, 0, 0))
    spec_art = pl.BlockSpec((1, H, HD), lambda b, c: (c, 0, 0))
    spec_b = pl.BlockSpec((1, 1, HD), lambda b, c: (c, 0, 0))
    spec_z_out = pl.BlockSpec((1, 1, N, HD), lambda b, c: (b, c, 0, 0))
    spec_s_out = pl.BlockSpec((1, 1, 1, 128), lambda b, c: (b, c, 0, 0))

    def full(shape):
        nd = len(shape)
        return pl.BlockSpec(shape, lambda b, c, _n=nd: (0,) * _n)

    z_shape = jax.ShapeDtypeStruct((B, C, N, HD), f32)
    s_shape = jax.ShapeDtypeStruct((B, C, 1, 128), f32)

    z0, s0 = pl.pallas_call(
        _k1_body,
        grid=(B, C),
        in_specs=[
            pl.BlockSpec((1, N, FIN), lambda b, c: (b, 0, 0)),
            spec_adj, spec_vc, spec_vr,
            pl.BlockSpec((1, FIN, HD), lambda b, c: (c, 0, 0)),
            spec_al, spec_art, spec_b,
            full((HD, 128)), full((1, 128)), full((128, 1)),
        ],
        out_specs=[spec_z_out, spec_s_out],
        out_shape=[z_shape, s_shape],
        compiler_params=pltpu.CompilerParams(
            dimension_semantics=("parallel", "arbitrary")),
    )(h0, adj, vc, vr, W0, al0m, ar0t, b0r, Wp1_0, bp1_0r, Wp2_0)

    beta0 = jax.nn.softmax(jnp.sum(s0[:, :, 0, 0], axis=0) / cnt)

    z1, s1 = pl.pallas_call(
        _k2_body,
        grid=(B, C),
        in_specs=[
            pl.BlockSpec((1, C, N, HD), lambda b, c: (b, 0, 0, 0)),
            pl.BlockSpec(memory_space=pltpu.SMEM),
            spec_adj, spec_vc, spec_vr,
            pl.BlockSpec((1, HD, HD), lambda b, c: (c, 0, 0)),
            spec_al, spec_art, spec_b,
            full((HD, 128)), full((1, 128)), full((128, 1)),
        ],
        out_specs=[spec_z_out, spec_s_out],
        out_shape=[z_shape, s_shape],
        scratch_shapes=[pltpu.VMEM((N, HD), f32)],
        compiler_params=pltpu.CompilerParams(
            dimension_semantics=("parallel", "arbitrary")),
    )(z0, beta0, adj, vc, vr, W1, al1m, ar1t, b1r, Wp1_1, bp1_1r, Wp2_1)

    beta1 = jax.nn.softmax(jnp.sum(s1[:, :, 0, 0], axis=0) / cnt)

    logits = pl.pallas_call(
        _k3_body,
        grid=(B,),
        in_specs=[
            pl.BlockSpec((1, C, N, HD), lambda b: (b, 0, 0, 0)),
            pl.BlockSpec(memory_space=pltpu.SMEM),
            pl.BlockSpec((HD, OUT), lambda b: (0, 0)),
            pl.BlockSpec((1, OUT), lambda b: (0, 0)),
            pl.BlockSpec((1, N, 1), lambda b: (b, 0, 0)),
        ],
        out_specs=pl.BlockSpec((1, N, OUT), lambda b: (b, 0, 0)),
        out_shape=jax.ShapeDtypeStruct((B, N, OUT), f32),
        compiler_params=pltpu.CompilerParams(
            dimension_semantics=("parallel",)),
    )(z1, beta1, Wpred, bpredr, vc)

    return logits


# speedup vs baseline: 1.9024x; 1.9024x over previous
import functools

import jax
import jax.numpy as jnp
from jax import lax
from jax.experimental import pallas as pl
from jax.experimental.pallas import tpu as pltpu

B, C, N, FIN = 4, 5, 512, 128
H, D = 4, 64
HD = H * D
OUT = 16
NEG = -1e9
_PREC = None


def _gat_core(h, adjm, vc, vr, W, Al, ArT, brow, Wp1, bp1, Wp2):
    feat = jnp.dot(h, W, precision=_PREC)
    el = jnp.dot(feat, Al, precision=_PREC)
    ert = lax.dot_general(ArT, feat, (((1,), (1,)), ((), ())),
                          precision=_PREC)
    cond = jnp.logical_and(adjm != 0.0,
                           jnp.logical_and(vc > 0.0, vr > 0.0))
    outs, denoms = [], []
    for hh in range(H):
        s = el[:, hh:hh + 1] + ert[hh:hh + 1, :]
        s = jnp.where(s >= 0.0, s, 0.2 * s)
        s = jnp.where(cond, s, NEG)
        m = jnp.max(s, axis=0, keepdims=True)
        p = jnp.exp(s - m)
        denoms.append(jnp.sum(p, axis=0, keepdims=True))
        outs.append(lax.dot_general(p, feat[:, hh * D:(hh + 1) * D],
                                    (((0,), (0,)), ((), ())),
                                    precision=_PREC))
    dnt = jnp.transpose(jnp.concatenate(denoms, axis=0))
    outs = [outs[hh] / dnt[:, hh:hh + 1] for hh in range(H)]
    rst = jnp.concatenate(outs, axis=1) + brow
    z = jnp.where(rst > 0.0, rst, jnp.exp(rst) - 1.0)
    p1 = jnp.tanh(jnp.dot(z, Wp1, precision=_PREC) + bp1)
    pr = jnp.dot(p1, Wp2, precision=_PREC)
    s_partial = jnp.sum(pr * vc)
    return z, s_partial


def _k1_body(h_ref, adj_ref, vc_ref, vr_ref, w_ref, al_ref, art_ref, b_ref,
             wp1_ref, bp1_ref, wp2_ref, z_ref, s_ref):
    z, s = _gat_core(h_ref[0], adj_ref[0, 0], vc_ref[0], vr_ref[0],
                     w_ref[0], al_ref[0], art_ref[0], b_ref[0],
                     wp1_ref[...], bp1_ref[...], wp2_ref[...])
    z_ref[0, 0] = z
    s_ref[0, 0, 0, :] = jnp.full((128,), s, dtype=jnp.float32)


def _k2_body(z0_ref, beta_ref, adj_ref, vc_ref, vr_ref, w_ref, al_ref,
             art_ref, b_ref, wp1_ref, bp1_ref, wp2_ref, z_ref, s_ref, h_scr):
    @pl.when(pl.program_id(1) == 0)
    def _combine():
        acc = beta_ref[0] * z0_ref[0, 0]
        for cc in range(1, C):
            acc = acc + beta_ref[cc] * z0_ref[0, cc]
        h_scr[...] = acc

    z, s = _gat_core(h_scr[...], adj_ref[0, 0], vc_ref[0], vr_ref[0],
                     w_ref[0], al_ref[0], art_ref[0], b_ref[0],
                     wp1_ref[...], bp1_ref[...], wp2_ref[...])
    z_ref[0, 0] = z
    s_ref[0, 0, 0, :] = jnp.full((128,), s, dtype=jnp.float32)


def _k3_body(z1_ref, beta_ref, wpred_ref, bpred_ref, vc_ref, out_ref):
    acc = beta_ref[0] * z1_ref[0, 0]
    for cc in range(1, C):
        acc = acc + beta_ref[cc] * z1_ref[0, cc]
    res = jnp.dot(acc, wpred_ref[...], precision=_PREC) + bpred_ref[...]
    out_ref[0] = res * vc_ref[0]


def _expand_attn(a):
    eye = jnp.eye(H, dtype=a.dtype)
    m = a[:, :, :, None] * eye[None, :, None, :]
    return m.reshape(C, HD, H)


def kernel(x, adj, node_nums, W0, al0, ar0, b0, Wp1_0, bp1_0, Wp2_0,
           W1, al1, ar1, b1, Wp1_1, bp1_1, Wp2_1, Wpred, bpred):
    f32 = jnp.float32
    h0 = x[:, 0]
    ar = jnp.arange(N, dtype=jnp.int32)
    valid = (ar[None, :] < node_nums[:, None]).astype(f32)
    vc = valid[:, :, None]
    vr = valid[:, None, :]
    cnt = jnp.maximum(jnp.sum(valid), 1.0)

    al0m, ar0t = _expand_attn(al0), _expand_attn(ar0).transpose(0, 2, 1)
    al1m, ar1t = _expand_attn(al1), _expand_attn(ar1).transpose(0, 2, 1)
    b0r, b1r = b0.reshape(C, 1, HD), b1.reshape(C, 1, HD)
    bp1_0r, bp1_1r = bp1_0.reshape(1, 128), bp1_1.reshape(1, 128)
    bpredr = bpred.reshape(1, OUT)

    spec_adj = pl.BlockSpec((1, 1, N, N), lambda b, c: (b, c, 0, 0))
    spec_vc = pl.BlockSpec((1, N, 1), lambda b, c: (b, 0, 0))
    spec_vr = pl.BlockSpec((1, 1, N), lambda b, c: (b, 0, 0))
    spec_al = pl.BlockSpec((1, HD, H), lambda b, c: (c, 0, 0))
    spec_art = pl.BlockSpec((1, H, HD), lambda b, c: (c, 0, 0))
    spec_b = pl.BlockSpec((1, 1, HD), lambda b, c: (c, 0, 0))
    spec_z_out = pl.BlockSpec((1, 1, N, HD), lambda b, c: (b, c, 0, 0))
    spec_s_out = pl.BlockSpec((1, 1, 1, 128), lambda b, c: (b, c, 0, 0))

    def full(shape):
        nd = len(shape)
        return pl.BlockSpec(shape, lambda b, c, _n=nd: (0,) * _n)

    z_shape = jax.ShapeDtypeStruct((B, C, N, HD), f32)
    s_shape = jax.ShapeDtypeStruct((B, C, 1, 128), f32)

    z0, s0 = pl.pallas_call(
        _k1_body,
        grid=(B, C),
        in_specs=[
            pl.BlockSpec((1, N, FIN), lambda b, c: (b, 0, 0)),
            spec_adj, spec_vc, spec_vr,
            pl.BlockSpec((1, FIN, HD), lambda b, c: (c, 0, 0)),
            spec_al, spec_art, spec_b,
            full((HD, 128)), full((1, 128)), full((128, 1)),
        ],
        out_specs=[spec_z_out, spec_s_out],
        out_shape=[z_shape, s_shape],
        compiler_params=pltpu.CompilerParams(
            dimension_semantics=("parallel", "arbitrary")),
    )(h0, adj, vc, vr, W0, al0m, ar0t, b0r, Wp1_0, bp1_0r, Wp2_0)

    beta0 = jax.nn.softmax(jnp.sum(s0[:, :, 0, 0], axis=0) / cnt)

    z1, s1 = pl.pallas_call(
        _k2_body,
        grid=(B, C),
        in_specs=[
            pl.BlockSpec((1, C, N, HD), lambda b, c: (b, 0, 0, 0)),
            pl.BlockSpec(memory_space=pltpu.SMEM),
            spec_adj, spec_vc, spec_vr,
            pl.BlockSpec((1, HD, HD), lambda b, c: (c, 0, 0)),
            spec_al, spec_art, spec_b,
            full((HD, 128)), full((1, 128)), full((128, 1)),
        ],
        out_specs=[spec_z_out, spec_s_out],
        out_shape=[z_shape, s_shape],
        scratch_shapes=[pltpu.VMEM((N, HD), f32)],
        compiler_params=pltpu.CompilerParams(
            dimension_semantics=("parallel", "arbitrary")),
    )(z0, beta0, adj, vc, vr, W1, al1m, ar1t, b1r, Wp1_1, bp1_1r, Wp2_1)

    beta1 = jax.nn.softmax(jnp.sum(s1[:, :, 0, 0], axis=0) / cnt)

    logits = pl.pallas_call(
        _k3_body,
        grid=(B,),
        in_specs=[
            pl.BlockSpec((1, C, N, HD), lambda b: (b, 0, 0, 0)),
            pl.BlockSpec(memory_space=pltpu.SMEM),
            pl.BlockSpec((HD, OUT), lambda b: (0, 0)),
            pl.BlockSpec((1, OUT), lambda b: (0, 0)),
            pl.BlockSpec((1, N, 1), lambda b: (b, 0, 0)),
        ],
        out_specs=pl.BlockSpec((1, N, OUT), lambda b: (b, 0, 0)),
        out_shape=jax.ShapeDtypeStruct((B, N, OUT), f32),
        compiler_params=pltpu.CompilerParams(
            dimension_semantics=("parallel",)),
    )(z1, beta1, Wpred, bpredr, vc)

    return logits


# speedup vs baseline: 2.4751x; 1.3011x over previous
import functools

import jax
import jax.numpy as jnp
from jax import lax
from jax.experimental import pallas as pl
from jax.experimental.pallas import tpu as pltpu

B, C, N, FIN = 4, 5, 512, 128
H, D = 4, 64
HD = H * D
OUT = 16
NEG = -1e9
_PREC = None


def _gat_core(h, adjm, vc, vr, W, Al, ArT, brow, Wp1, bp1, Wp2):
    feat = jnp.dot(h, W, precision=_PREC)
    el = jnp.dot(feat, Al, precision=_PREC)
    ert = lax.dot_general(ArT, feat, (((1,), (1,)), ((), ())),
                          precision=_PREC)
    cond = jnp.logical_and(adjm != 0.0, vc > 0.0)
    ones_col = jnp.ones((N, 1), dtype=jnp.float32)
    outs = []
    for hh in range(H):
        s = el[:, hh:hh + 1] + ert[hh:hh + 1, :]
        s = jnp.maximum(s, 0.2 * s)
        p = lax.exp2(jnp.where(cond, s, -43.2808512))
        fe = jnp.concatenate([feat[:, hh * D:(hh + 1) * D], ones_col], axis=1)
        oe = lax.dot_general(p, fe, (((0,), (0,)), ((), ())),
                             precision=_PREC)
        outs.append(oe[:, :D] / oe[:, D:D + 1])
    rst = jnp.concatenate(outs, axis=1) + brow
    z = jnp.where(rst > 0.0, rst, jnp.exp(rst) - 1.0)
    p1 = jnp.tanh(jnp.dot(z, Wp1, precision=_PREC) + bp1)
    pr = jnp.dot(p1, Wp2, precision=_PREC)
    s_partial = jnp.sum(pr * vc)
    return z, s_partial


def _k1_body(h_ref, adj_ref, vc_ref, vr_ref, w_ref, al_ref, art_ref, b_ref,
             wp1_ref, bp1_ref, wp2_ref, z_ref, s_ref):
    z, s = _gat_core(h_ref[0], adj_ref[0, 0], vc_ref[0], vr_ref[0],
                     w_ref[0], al_ref[0], art_ref[0], b_ref[0],
                     wp1_ref[...], bp1_ref[...], wp2_ref[...])
    z_ref[0, 0] = z
    s_ref[0, 0, 0, :] = jnp.full((128,), s, dtype=jnp.float32)


def _k2_body(z0_ref, beta_ref, adj_ref, vc_ref, vr_ref, w_ref, al_ref,
             art_ref, b_ref, wp1_ref, bp1_ref, wp2_ref, z_ref, s_ref, h_scr):
    @pl.when(pl.program_id(1) == 0)
    def _combine():
        acc = beta_ref[0] * z0_ref[0, 0]
        for cc in range(1, C):
            acc = acc + beta_ref[cc] * z0_ref[0, cc]
        h_scr[...] = acc

    z, s = _gat_core(h_scr[...], adj_ref[0, 0], vc_ref[0], vr_ref[0],
                     w_ref[0], al_ref[0], art_ref[0], b_ref[0],
                     wp1_ref[...], bp1_ref[...], wp2_ref[...])
    z_ref[0, 0] = z
    s_ref[0, 0, 0, :] = jnp.full((128,), s, dtype=jnp.float32)


def _k3_body(z1_ref, beta_ref, wpred_ref, bpred_ref, vc_ref, out_ref):
    acc = beta_ref[0] * z1_ref[0, 0]
    for cc in range(1, C):
        acc = acc + beta_ref[cc] * z1_ref[0, cc]
    res = jnp.dot(acc, wpred_ref[...], precision=_PREC) + bpred_ref[...]
    out_ref[0] = res * vc_ref[0]


def _expand_attn(a):
    eye = jnp.eye(H, dtype=a.dtype)
    m = a[:, :, :, None] * eye[None, :, None, :]
    return m.reshape(C, HD, H)


def kernel(x, adj, node_nums, W0, al0, ar0, b0, Wp1_0, bp1_0, Wp2_0,
           W1, al1, ar1, b1, Wp1_1, bp1_1, Wp2_1, Wpred, bpred):
    f32 = jnp.float32
    h0 = x[:, 0]
    ar = jnp.arange(N, dtype=jnp.int32)
    valid = (ar[None, :] < node_nums[:, None]).astype(f32)
    vc = valid[:, :, None]
    vr = valid[:, None, :]
    cnt = jnp.maximum(jnp.sum(valid), 1.0)

    log2e = jnp.float32(1.4426950408889634)
    al0m, ar0t = _expand_attn(al0) * log2e, _expand_attn(ar0).transpose(0, 2, 1) * log2e
    al1m, ar1t = _expand_attn(al1) * log2e, _expand_attn(ar1).transpose(0, 2, 1) * log2e
    b0r, b1r = b0.reshape(C, 1, HD), b1.reshape(C, 1, HD)
    bp1_0r, bp1_1r = bp1_0.reshape(1, 128), bp1_1.reshape(1, 128)
    bpredr = bpred.reshape(1, OUT)

    spec_adj = pl.BlockSpec((1, 1, N, N), lambda b, c: (b, c, 0, 0))
    spec_vc = pl.BlockSpec((1, N, 1), lambda b, c: (b, 0, 0))
    spec_vr = pl.BlockSpec((1, 1, N), lambda b, c: (b, 0, 0))
    spec_al = pl.BlockSpec((1, HD, H), lambda b, c: (c, 0, 0))
    spec_art = pl.BlockSpec((1, H, HD), lambda b, c: (c, 0, 0))
    spec_b = pl.BlockSpec((1, 1, HD), lambda b, c: (c, 0, 0))
    spec_z_out = pl.BlockSpec((1, 1, N, HD), lambda b, c: (b, c, 0, 0))
    spec_s_out = pl.BlockSpec((1, 1, 1, 128), lambda b, c: (b, c, 0, 0))

    def full(shape):
        nd = len(shape)
        return pl.BlockSpec(shape, lambda b, c, _n=nd: (0,) * _n)

    z_shape = jax.ShapeDtypeStruct((B, C, N, HD), f32)
    s_shape = jax.ShapeDtypeStruct((B, C, 1, 128), f32)

    z0, s0 = pl.pallas_call(
        _k1_body,
        grid=(B, C),
        in_specs=[
            pl.BlockSpec((1, N, FIN), lambda b, c: (b, 0, 0)),
            spec_adj, spec_vc, spec_vr,
            pl.BlockSpec((1, FIN, HD), lambda b, c: (c, 0, 0)),
            spec_al, spec_art, spec_b,
            full((HD, 128)), full((1, 128)), full((128, 1)),
        ],
        out_specs=[spec_z_out, spec_s_out],
        out_shape=[z_shape, s_shape],
        compiler_params=pltpu.CompilerParams(
            dimension_semantics=("parallel", "arbitrary")),
    )(h0, adj, vc, vr, W0, al0m, ar0t, b0r, Wp1_0, bp1_0r, Wp2_0)

    beta0 = jax.nn.softmax(jnp.sum(s0[:, :, 0, 0], axis=0) / cnt)

    z1, s1 = pl.pallas_call(
        _k2_body,
        grid=(B, C),
        in_specs=[
            pl.BlockSpec((1, C, N, HD), lambda b, c: (b, 0, 0, 0)),
            pl.BlockSpec(memory_space=pltpu.SMEM),
            spec_adj, spec_vc, spec_vr,
            pl.BlockSpec((1, HD, HD), lambda b, c: (c, 0, 0)),
            spec_al, spec_art, spec_b,
            full((HD, 128)), full((1, 128)), full((128, 1)),
        ],
        out_specs=[spec_z_out, spec_s_out],
        out_shape=[z_shape, s_shape],
        scratch_shapes=[pltpu.VMEM((N, HD), f32)],
        compiler_params=pltpu.CompilerParams(
            dimension_semantics=("parallel", "arbitrary")),
    )(z0, beta0, adj, vc, vr, W1, al1m, ar1t, b1r, Wp1_1, bp1_1r, Wp2_1)

    beta1 = jax.nn.softmax(jnp.sum(s1[:, :, 0, 0], axis=0) / cnt)

    logits = pl.pallas_call(
        _k3_body,
        grid=(B,),
        in_specs=[
            pl.BlockSpec((1, C, N, HD), lambda b: (b, 0, 0, 0)),
            pl.BlockSpec(memory_space=pltpu.SMEM),
            pl.BlockSpec((HD, OUT), lambda b: (0, 0)),
            pl.BlockSpec((1, OUT), lambda b: (0, 0)),
            pl.BlockSpec((1, N, 1), lambda b: (b, 0, 0)),
        ],
        out_specs=pl.BlockSpec((1, N, OUT), lambda b: (b, 0, 0)),
        out_shape=jax.ShapeDtypeStruct((B, N, OUT), f32),
        compiler_params=pltpu.CompilerParams(
            dimension_semantics=("parallel",)),
    )(z1, beta1, Wpred, bpredr, vc)

    return logits


# speedup vs baseline: 2.6245x; 1.0603x over previous
import jax
import jax.numpy as jnp
from jax import lax
from jax.experimental import pallas as pl
from jax.experimental.pallas import tpu as pltpu

B, C, N, FIN = 4, 5, 512, 128
H, D = 4, 64
HD = H * D
OUT = 16
P0, P1, P2 = B * C, 2 * B * C, 2 * B * C + B
LOG2E = 1.4426950408889634
_PREC = None


def _gat_core(h, adjm, vc, W, Al, ArT, brow, Wp1, bp1, Wp2):
    feat = jnp.dot(h, W, precision=_PREC)
    el = jnp.dot(feat, Al, precision=_PREC)
    ert = lax.dot_general(ArT, feat, (((1,), (1,)), ((), ())),
                          precision=_PREC)
    cond = jnp.logical_and(adjm != 0.0, vc > 0.0)
    ones_col = jnp.ones((N, 1), dtype=jnp.float32)
    outs = []
    for hh in range(H):
        s = el[:, hh:hh + 1] + ert[hh:hh + 1, :]
        s = jnp.maximum(s, 0.2 * s)
        p = lax.exp2(jnp.where(cond, s, -43.2808512))
        fe = jnp.concatenate([feat[:, hh * D:(hh + 1) * D], ones_col], axis=1)
        oe = lax.dot_general(p, fe, (((0,), (0,)), ((), ())),
                             precision=_PREC)
        outs.append(oe[:, :D] / oe[:, D:D + 1])
    rst = jnp.concatenate(outs, axis=1) + brow
    z = jnp.where(rst > 0.0, rst, jnp.exp(rst) - 1.0)
    p1 = jnp.tanh(jnp.dot(z, Wp1, precision=_PREC) + bp1)
    pr = jnp.dot(p1, Wp2, precision=_PREC)
    s_partial = jnp.sum(pr * vc)
    return z, s_partial


def _beta_tile(s_tile, inv_cnt):
    rows = lax.broadcasted_iota(jnp.int32, (8, 128), 0)
    t = jnp.where(rows < C, s_tile * inv_cnt, -1e30)
    m = jnp.max(t, axis=0, keepdims=True)
    e = lax.exp2((t - m) * LOG2E)
    return e / jnp.sum(e, axis=0, keepdims=True)


def _body(x_ref, adj_ref, vc_ref, cnt_ref,
          w0_ref, al0_ref, art0_ref, b0_ref, wp10_ref, bp10_ref, wp20_ref,
          w1_ref, al1_ref, art1_ref, b1_ref, wp11_ref, bp11_ref, wp21_ref,
          wpred_ref, bpred_ref, out_ref,
          z0_scr, z1_scr, h_scr, s_scr, beta_scr):
    i = pl.program_id(0)
    rows = lax.broadcasted_iota(jnp.int32, (8, 128), 0)

    @pl.when(i == 0)
    def _init():
        s_scr[...] = jnp.zeros((2, 8, 128), dtype=jnp.float32)

    @pl.when(i < P0)
    def _layer0():
        b, c = i // C, i % C
        z, sp = _gat_core(x_ref[0], adj_ref[0, 0], vc_ref[0],
                          w0_ref[0], al0_ref[0], art0_ref[0], b0_ref[0],
                          wp10_ref[...], bp10_ref[...], wp20_ref[...])
        z0_scr[b, c] = z
        s_scr[0] = s_scr[0] + jnp.where(rows == c, sp, 0.0)

    @pl.when(i == P0)
    def _beta0():
        beta_scr[0] = _beta_tile(s_scr[0], 1.0 / cnt_ref[0])

    @pl.when(jnp.logical_and(i >= P0, i < P1))
    def _layer1():
        j = i - P0
        b, c = j // C, j % C

        @pl.when(c == 0)
        def _combine():
            acc = z0_scr[b, 0] * beta_scr[0, 0:1, 0:1]
            for cc in range(1, C):
                acc = acc + z0_scr[b, cc] * beta_scr[0, cc:cc + 1, 0:1]
            h_scr[...] = acc

        z, sp = _gat_core(h_scr[...], adj_ref[0, 0], vc_ref[0],
                          w1_ref[0], al1_ref[0], art1_ref[0], b1_ref[0],
                          wp11_ref[...], bp11_ref[...], wp21_ref[...])
        z1_scr[b, c] = z
        s_scr[1] = s_scr[1] + jnp.where(rows == c, sp, 0.0)

    @pl.when(i == P1)
    def _beta1():
        beta_scr[1] = _beta_tile(s_scr[1], 1.0 / cnt_ref[0])

    @pl.when(i >= P1)
    def _predict():
        b = i - P1
        acc = z1_scr[b, 0] * beta_scr[1, 0:1, 0:1]
        for cc in range(1, C):
            acc = acc + z1_scr[b, cc] * beta_scr[1, cc:cc + 1, 0:1]
        res = jnp.dot(acc, wpred_ref[...], precision=_PREC) + bpred_ref[...]
        out_ref[0] = res * vc_ref[0]


def _expand_attn(a):
    eye = jnp.eye(H, dtype=a.dtype)
    m = a[:, :, :, None] * eye[None, :, None, :]
    return m.reshape(C, HD, H)


def kernel(x, adj, node_nums, W0, al0, ar0, b0, Wp1_0, bp1_0, Wp2_0,
           W1, al1, ar1, b1, Wp1_1, bp1_1, Wp2_1, Wpred, bpred):
    f32 = jnp.float32
    h0 = x[:, 0]
    ar = jnp.arange(N, dtype=jnp.int32)
    valid = (ar[None, :] < node_nums[:, None]).astype(f32)
    vc = valid[:, :, None]
    cnt = jnp.maximum(jnp.sum(valid), 1.0).reshape(1)

    log2e = jnp.float32(LOG2E)
    al0m, ar0t = _expand_attn(al0) * log2e, _expand_attn(ar0).transpose(0, 2, 1) * log2e
    al1m, ar1t = _expand_attn(al1) * log2e, _expand_attn(ar1).transpose(0, 2, 1) * log2e
    b0r, b1r = b0.reshape(C, 1, HD), b1.reshape(C, 1, HD)
    bp1_0r, bp1_1r = bp1_0.reshape(1, 128), bp1_1.reshape(1, 128)
    bpredr = bpred.reshape(1, OUT)

    def jmap(i):
        return jnp.where(i >= P1, (i - P1) * C, jnp.where(i >= P0, i - P0, i))

    def bidx(i):
        return jmap(i) // C

    def cidx(i):
        return jmap(i) % C

    def full(shape):
        nd = len(shape)
        return pl.BlockSpec(shape, lambda i, _n=nd: (0,) * _n)

    logits = pl.pallas_call(
        _body,
        grid=(P2,),
        in_specs=[
            pl.BlockSpec((1, N, FIN), lambda i: (bidx(i) % B, 0, 0)),
            pl.BlockSpec((1, 1, N, N), lambda i: (bidx(i), cidx(i), 0, 0)),
            pl.BlockSpec((1, N, 1), lambda i: (bidx(i), 0, 0)),
            pl.BlockSpec(memory_space=pltpu.SMEM),
            pl.BlockSpec((1, FIN, HD), lambda i: (cidx(i), 0, 0)),
            pl.BlockSpec((1, HD, H), lambda i: (cidx(i), 0, 0)),
            pl.BlockSpec((1, H, HD), lambda i: (cidx(i), 0, 0)),
            pl.BlockSpec((1, 1, HD), lambda i: (cidx(i), 0, 0)),
            full((HD, 128)), full((1, 128)), full((128, 1)),
            pl.BlockSpec((1, HD, HD), lambda i: (cidx(i), 0, 0)),
            pl.BlockSpec((1, HD, H), lambda i: (cidx(i), 0, 0)),
            pl.BlockSpec((1, H, HD), lambda i: (cidx(i), 0, 0)),
            pl.BlockSpec((1, 1, HD), lambda i: (cidx(i), 0, 0)),
            full((HD, 128)), full((1, 128)), full((128, 1)),
            full((HD, OUT)), full((1, OUT)),
        ],
        out_specs=pl.BlockSpec(
            (1, N, OUT), lambda i: (jnp.where(i >= P1, i - P1, 0), 0, 0)),
        out_shape=jax.ShapeDtypeStruct((B, N, OUT), f32),
        scratch_shapes=[
            pltpu.VMEM((B, C, N, HD), f32),
            pltpu.VMEM((B, C, N, HD), f32),
            pltpu.VMEM((N, HD), f32),
            pltpu.VMEM((2, 8, 128), f32),
            pltpu.VMEM((2, 8, 128), f32),
        ],
        compiler_params=pltpu.CompilerParams(
            dimension_semantics=("arbitrary",)),
    )(h0, adj, vc, cnt,
      W0, al0m, ar0t, b0r, Wp1_0, bp1_0r, Wp2_0,
      W1, al1m, ar1t, b1r, Wp1_1, bp1_1r, Wp2_1,
      Wpred, bpredr)

    return logits


# speedup vs baseline: 2.7048x; 1.0306x over previous
import jax
import jax.numpy as jnp
from jax import lax
from jax.experimental import pallas as pl
from jax.experimental.pallas import tpu as pltpu

B, C, N, FIN = 4, 5, 512, 128
H, D = 4, 64
HD = H * D
OUT = 16
P0, P1, P2 = B * C, 2 * B * C, 2 * B * C + B
LOG2E = 1.4426950408889634
_PREC = None


def _gat_core(h, adjm, nn, W, Al, ArT, brow, Wp1, bp1, Wp2):
    iota_col = lax.broadcasted_iota(jnp.int32, (N, 1), 0)
    vcf = (iota_col < nn).astype(jnp.float32)
    feat = jnp.dot(h, W, precision=_PREC)
    el = jnp.dot(feat, Al, precision=_PREC)
    ert = lax.dot_general(ArT, feat, (((1,), (1,)), ((), ())),
                          precision=_PREC)
    cond = jnp.logical_and(adjm != 0.0, iota_col < nn)
    ones_col = jnp.ones((N, 1), dtype=jnp.float32)
    outs = []
    for hh in range(H):
        fh = feat[:, hh * D:(hh + 1) * D]
        s = el[:, hh:hh + 1] + ert[hh:hh + 1, :]
        s = jnp.maximum(s, 0.2 * s)
        p = lax.exp2(jnp.where(cond, s, -43.2808512))
        fe = jnp.concatenate([fh, ones_col], axis=1)
        oe = lax.dot_general(p, fe, (((0,), (0,)), ((), ())),
                             precision=_PREC)
        outs.append(oe[:, :D] * (1.0 / oe[:, D:D + 1]))
    rst = jnp.concatenate(outs, axis=1) + brow
    z = jnp.where(rst > 0.0, rst, jnp.exp(rst) - 1.0)
    p1 = jnp.tanh(jnp.dot(z, Wp1, precision=_PREC) + bp1)
    pr = jnp.dot(p1, Wp2, precision=_PREC)
    s_partial = jnp.sum(pr * vcf)
    return z, s_partial


def _beta_tile(s_tile, cnt):
    rows = lax.broadcasted_iota(jnp.int32, (8, 128), 0)
    t = jnp.where(rows < C, s_tile / cnt, -1e30)
    m = jnp.max(t, axis=0, keepdims=True)
    e = lax.exp2((t - m) * LOG2E)
    return e / jnp.sum(e, axis=0, keepdims=True)


def _body(x_ref, adj_ref, nn_ref,
          w0_ref, al0_ref, ar0_ref, b0_ref, wp10_ref, bp10_ref, wp20_ref,
          w1_ref, al1_ref, ar1_ref, b1_ref, wp11_ref, bp11_ref, wp21_ref,
          wpred_ref, bpred_ref, out_ref,
          z0_scr, z1_scr, h_scr, s_scr, beta_scr):
    i = pl.program_id(0)
    rows = lax.broadcasted_iota(jnp.int32, (8, 128), 0)
    cnt = jnp.maximum(
        (nn_ref[0] + nn_ref[1] + nn_ref[2] + nn_ref[3]).astype(jnp.float32),
        1.0)

    @pl.when(i == 0)
    def _init():
        s_scr[...] = jnp.zeros((2, 8, 128), dtype=jnp.float32)

    @pl.when(i < P0)
    def _layer0():
        b, c = i // C, i % C
        z, sp = _gat_core(x_ref[0, 0], adj_ref[0, 0], nn_ref[b],
                          w0_ref[0], al0_ref[0], ar0_ref[0], b0_ref[0],
                          wp10_ref[...], bp10_ref[...], wp20_ref[...])
        z0_scr[b, c] = z
        s_scr[0] = s_scr[0] + jnp.where(rows == c, sp, 0.0)

    @pl.when(i == P0)
    def _beta0():
        beta_scr[0] = _beta_tile(s_scr[0], cnt)

    @pl.when(jnp.logical_and(i >= P0, i < P1))
    def _layer1():
        j = i - P0
        b, c = j // C, j % C

        @pl.when(c == 0)
        def _combine():
            acc = z0_scr[b, 0] * beta_scr[0, 0:1, 0:1]
            for cc in range(1, C):
                acc = acc + z0_scr[b, cc] * beta_scr[0, cc:cc + 1, 0:1]
            h_scr[...] = acc

        z, sp = _gat_core(h_scr[...], adj_ref[0, 0], nn_ref[b],
                          w1_ref[0], al1_ref[0], ar1_ref[0], b1_ref[0],
                          wp11_ref[...], bp11_ref[...], wp21_ref[...])
        z1_scr[b, c] = z
        s_scr[1] = s_scr[1] + jnp.where(rows == c, sp, 0.0)

    @pl.when(i == P1)
    def _beta1():
        beta_scr[1] = _beta_tile(s_scr[1], cnt)

    @pl.when(i >= P1)
    def _predict():
        b = i - P1
        acc = z1_scr[b, 0] * beta_scr[1, 0:1, 0:1]
        for cc in range(1, C):
            acc = acc + z1_scr[b, cc] * beta_scr[1, cc:cc + 1, 0:1]
        res = jnp.dot(acc, wpred_ref[...], precision=_PREC) + bpred_ref[...]
        vcf = (lax.broadcasted_iota(jnp.int32, (N, 1), 0)
               < nn_ref[b]).astype(jnp.float32)
        out_ref[0] = res * vcf


def _expand_attn(a):
    eye = jnp.eye(H, dtype=a.dtype)
    m = a[:, :, :, None] * eye[None, :, None, :]
    return m.reshape(C, HD, H)


def kernel(x, adj, node_nums, W0, al0, ar0, b0, Wp1_0, bp1_0, Wp2_0,
           W1, al1, ar1, b1, Wp1_1, bp1_1, Wp2_1, Wpred, bpred):
    f32 = jnp.float32
    nn32 = node_nums.astype(jnp.int32)
    log2e = jnp.float32(LOG2E)
    al0m = _expand_attn(al0) * log2e
    ar0t = _expand_attn(ar0).transpose(0, 2, 1) * log2e
    al1m = _expand_attn(al1) * log2e
    ar1t = _expand_attn(ar1).transpose(0, 2, 1) * log2e
    b0r, b1r = b0.reshape(C, 1, HD), b1.reshape(C, 1, HD)
    bp1_0r, bp1_1r = bp1_0.reshape(1, 128), bp1_1.reshape(1, 128)
    bpredr = bpred.reshape(1, OUT)

    def jmap(i):
        return jnp.where(i >= P1, (i - P1) * C, jnp.where(i >= P0, i - P0, i))

    def full(shape):
        nd = len(shape)
        return pl.BlockSpec(shape, lambda i, _n=nd: (0,) * _n)

    def c0idx(i):
        return jnp.where(i < P0, i % C, 0)

    def c1idx(i):
        return jnp.where(jnp.logical_and(i >= P0, i < P1), (i - P0) % C, 0)

    logits = pl.pallas_call(
        _body,
        grid=(P2,),
        in_specs=[
            pl.BlockSpec((1, 1, N, FIN),
                         lambda i: (jnp.where(i < P0, i // C, 0), 0, 0, 0)),
            pl.BlockSpec((1, 1, N, N),
                         lambda i: (jnp.where(i < P1, jmap(i) // C, 0),
                                    jnp.where(i < P1, jmap(i) % C, 0), 0, 0)),
            pl.BlockSpec(memory_space=pltpu.SMEM),
            pl.BlockSpec((1, FIN, HD), lambda i: (c0idx(i), 0, 0)),
            pl.BlockSpec((1, HD, H), lambda i: (c0idx(i), 0, 0)),
            pl.BlockSpec((1, H, HD), lambda i: (c0idx(i), 0, 0)),
            pl.BlockSpec((1, 1, HD), lambda i: (c0idx(i), 0, 0)),
            full((HD, 128)), full((1, 128)), full((128, 1)),
            pl.BlockSpec((1, HD, HD), lambda i: (c1idx(i), 0, 0)),
            pl.BlockSpec((1, HD, H), lambda i: (c1idx(i), 0, 0)),
            pl.BlockSpec((1, H, HD), lambda i: (c1idx(i), 0, 0)),
            pl.BlockSpec((1, 1, HD), lambda i: (c1idx(i), 0, 0)),
            full((HD, 128)), full((1, 128)), full((128, 1)),
            full((HD, OUT)), full((1, OUT)),
        ],
        out_specs=pl.BlockSpec(
            (1, N, OUT), lambda i: (jnp.where(i >= P1, i - P1, 0), 0, 0)),
        out_shape=jax.ShapeDtypeStruct((B, N, OUT), f32),
        scratch_shapes=[
            pltpu.VMEM((B, C, N, HD), f32),
            pltpu.VMEM((B, C, N, HD), f32),
            pltpu.VMEM((N, HD), f32),
            pltpu.VMEM((2, 8, 128), f32),
            pltpu.VMEM((2, 8, 128), f32),
        ],
        compiler_params=pltpu.CompilerParams(
            dimension_semantics=("arbitrary",)),
    )(x, adj, nn32,
      W0, al0m, ar0t, b0r, Wp1_0, bp1_0r, Wp2_0,
      W1, al1m, ar1t, b1r, Wp1_1, bp1_1r, Wp2_1,
      Wpred, bpredr)

    return logits
